# initial kernel scaffold (unmeasured)
import jax
import jax.numpy as jnp
from jax import lax
from jax.experimental import pallas as pl
from jax.experimental.pallas import tpu as pltpu

N_DEV = 16


def kernel(x, w_mat):
    m_per, k = x.shape
    _, n = w_mat.shape
    n_per = n // N_DEV

    def body(x_ref, w_ref, out_ref, y_ref, q_ref, recv_ref, amax_ref,
             amax_send_sems, amax_recv_sems, data_send_sems, data_recv_sems):
        my_i = lax.axis_index("i")

        y = jnp.maximum(
            jnp.dot(x_ref[...], w_ref[...], preferred_element_type=jnp.float32),
            0.0,
        )
        y_ref[...] = y
        local_amax = jnp.max(y)

        for s in range(N_DEV):
            @pl.when(s == my_i)
            def _():
                amax_ref[s] = jnp.full((8, 128), local_amax, jnp.float32)

        for s in range(N_DEV):
            @pl.when(s != my_i)
            def _():
                rdma = pltpu.make_async_remote_copy(
                    src_ref=amax_ref.at[my_i],
                    dst_ref=amax_ref.at[my_i],
                    send_sem=amax_send_sems.at[s],
                    recv_sem=amax_recv_sems.at[my_i],
                    device_id=(s,),
                    device_id_type=pl.DeviceIdType.MESH,
                )
                rdma.start()
        for s in range(N_DEV):
            @pl.when(s != my_i)
            def _():
                d = pltpu.make_async_remote_copy(
                    src_ref=amax_ref.at[s],
                    dst_ref=amax_ref.at[s],
                    send_sem=amax_send_sems.at[s],
                    recv_sem=amax_recv_sems.at[s],
                    device_id=(s,),
                    device_id_type=pl.DeviceIdType.MESH,
                )
                d.wait_recv()
                d.wait_send()

        gmax = jnp.max(amax_ref[...])
        scale = gmax / 127.0

        q_ref[...] = jnp.clip(
            jnp.round(y_ref[...] / scale), -127.0, 127.0
        ).astype(jnp.int8)

        recv_ref[pl.ds(my_i * m_per, m_per), :] = (
            q_ref[:, pl.ds(my_i * n_per, n_per)]
        )

        for s in range(N_DEV):
            @pl.when(s != my_i)
            def _():
                rdma = pltpu.make_async_remote_copy(
                    src_ref=q_ref.at[:, pl.ds(s * n_per, n_per)],
                    dst_ref=recv_ref.at[pl.ds(my_i * m_per, m_per), :],
                    send_sem=data_send_sems.at[s],
                    recv_sem=data_recv_sems.at[my_i],
                    device_id=(s,),
                    device_id_type=pl.DeviceIdType.MESH,
                )
                rdma.start()
        for s in range(N_DEV):
            @pl.when(s != my_i)
            def _():
                d = pltpu.make_async_remote_copy(
                    src_ref=q_ref.at[:, pl.ds(s * n_per, n_per)],
                    dst_ref=recv_ref.at[pl.ds(s * m_per, m_per), :],
                    send_sem=data_send_sems.at[s],
                    recv_sem=data_recv_sems.at[s],
                    device_id=(s,),
                    device_id_type=pl.DeviceIdType.MESH,
                )
                d.wait_recv()
                d.wait_send()

        out_ref[...] = recv_ref[...].astype(jnp.float32) * scale

    return pl.pallas_call(
        body,
        out_shape=jax.ShapeDtypeStruct((N_DEV * m_per, n_per), jnp.float32),
        in_specs=[
            pl.BlockSpec(memory_space=pltpu.VMEM),
            pl.BlockSpec(memory_space=pltpu.VMEM),
        ],
        out_specs=pl.BlockSpec(memory_space=pltpu.VMEM),
        scratch_shapes=[
            pltpu.VMEM((m_per, n), jnp.float32),
            pltpu.VMEM((m_per, n), jnp.int8),
            pltpu.VMEM((N_DEV * m_per, n_per), jnp.int8),
            pltpu.VMEM((N_DEV, 8, 128), jnp.float32),
            pltpu.SemaphoreType.DMA((N_DEV,)),
            pltpu.SemaphoreType.DMA((N_DEV,)),
            pltpu.SemaphoreType.DMA((N_DEV,)),
            pltpu.SemaphoreType.DMA((N_DEV,)),
        ],
    )(x, w_mat)


# baseline (device time: 41816 ns/iter reference)
import jax
import jax.numpy as jnp
from jax import lax
from jax.experimental import pallas as pl
from jax.experimental.pallas import tpu as pltpu

N_DEV = 16


def kernel(x, w_mat):
    m_per, k = x.shape
    _, n = w_mat.shape
    n_per = n // N_DEV

    def body(x_ref, w_ref, out_ref, y_ref, q_ref, recv_ref, amax_ref,
             amax_send_sems, amax_recv_sems, data_send_sems, data_recv_sems):
        my_i = lax.axis_index("i")

        y = jnp.maximum(
            jnp.dot(x_ref[...], w_ref[...], preferred_element_type=jnp.float32),
            0.0,
        )
        y_ref[...] = y
        local_amax = jnp.max(y)

        for s in range(N_DEV):
            @pl.when(s == my_i)
            def _():
                amax_ref[s] = jnp.full((8, 128), local_amax, jnp.float32)

        for s in range(N_DEV):
            @pl.when(s != my_i)
            def _():
                rdma = pltpu.make_async_remote_copy(
                    src_ref=amax_ref.at[my_i],
                    dst_ref=amax_ref.at[my_i],
                    send_sem=amax_send_sems.at[s],
                    recv_sem=amax_recv_sems.at[my_i],
                    device_id=(s,),
                    device_id_type=pl.DeviceIdType.MESH,
                )
                rdma.start()
        for s in range(N_DEV):
            @pl.when(s != my_i)
            def _():
                d = pltpu.make_async_remote_copy(
                    src_ref=amax_ref.at[s],
                    dst_ref=amax_ref.at[s],
                    send_sem=amax_send_sems.at[s],
                    recv_sem=amax_recv_sems.at[s],
                    device_id=(s,),
                    device_id_type=pl.DeviceIdType.MESH,
                )
                d.wait_recv()
                d.wait_send()

        gmax = jnp.max(amax_ref[...])
        scale = gmax / 127.0

        q_ref[...] = jnp.clip(
            jnp.round(y_ref[...] / scale), -127.0, 127.0
        ).astype(jnp.int8)

        recv_ref[pl.ds(my_i * m_per, m_per), :] = (
            q_ref[:, pl.ds(my_i * n_per, n_per)]
        )

        for s in range(N_DEV):
            @pl.when(s != my_i)
            def _():
                rdma = pltpu.make_async_remote_copy(
                    src_ref=q_ref.at[:, pl.ds(s * n_per, n_per)],
                    dst_ref=recv_ref.at[pl.ds(my_i * m_per, m_per), :],
                    send_sem=data_send_sems.at[s],
                    recv_sem=data_recv_sems.at[my_i],
                    device_id=(s,),
                    device_id_type=pl.DeviceIdType.MESH,
                )
                rdma.start()
        for s in range(N_DEV):
            @pl.when(s != my_i)
            def _():
                d = pltpu.make_async_remote_copy(
                    src_ref=q_ref.at[:, pl.ds(s * n_per, n_per)],
                    dst_ref=recv_ref.at[pl.ds(s * m_per, m_per), :],
                    send_sem=data_send_sems.at[s],
                    recv_sem=data_recv_sems.at[s],
                    device_id=(s,),
                    device_id_type=pl.DeviceIdType.MESH,
                )
                d.wait_recv()
                d.wait_send()

        out_ref[...] = recv_ref[...].astype(jnp.float32) * scale

    return pl.pallas_call(
        body,
        out_shape=jax.ShapeDtypeStruct((N_DEV * m_per, n_per), jnp.float32),
        in_specs=[
            pl.BlockSpec(memory_space=pltpu.VMEM),
            pl.BlockSpec(memory_space=pltpu.VMEM),
        ],
        out_specs=pl.BlockSpec(memory_space=pltpu.VMEM),
        scratch_shapes=[
            pltpu.VMEM((m_per, n), jnp.float32),
            pltpu.VMEM((m_per, n), jnp.int8),
            pltpu.VMEM((N_DEV * m_per, n_per), jnp.int8),
            pltpu.VMEM((N_DEV, 8, 128), jnp.float32),
            pltpu.SemaphoreType.DMA((N_DEV,)),
            pltpu.SemaphoreType.DMA((N_DEV,)),
            pltpu.SemaphoreType.DMA((N_DEV,)),
            pltpu.SemaphoreType.DMA((N_DEV,)),
        ],
        compiler_params=pltpu.CompilerParams(
            vmem_limit_bytes=100 * 1024 * 1024,
        ),
    )(x, w_mat)


# device time: 35417 ns/iter; 1.1807x vs baseline; 1.1807x over previous
import jax
import jax.numpy as jnp
from jax import lax
from jax.experimental import pallas as pl
from jax.experimental.pallas import tpu as pltpu

N_DEV = 16
N_CHUNKS = 4


def kernel(x, w_mat):
    m_per, k = x.shape
    _, n = w_mat.shape
    n_per = n // N_DEV
    cw = n // N_CHUNKS

    def body(x_ref, w_hbm, out_ref, wbuf, y_ref, q_ref, recv_ref, amax_ref,
             wdma_sems, amax_send_sems, amax_recv_sems,
             data_send_sems, data_recv_sems):
        my_i = lax.axis_index("i")

        barrier_sem = pltpu.get_barrier_semaphore()
        for s in range(N_DEV):
            @pl.when(s != my_i)
            def _():
                pl.semaphore_signal(
                    barrier_sem, inc=1,
                    device_id=(s,), device_id_type=pl.DeviceIdType.MESH,
                )
        pl.semaphore_wait(barrier_sem, N_DEV - 1)

        def w_copy(c, slot):
            return pltpu.make_async_copy(
                w_hbm.at[:, pl.ds(c * cw, cw)],
                wbuf.at[slot],
                wdma_sems.at[slot],
            )

        w_copy(0, 0).start()
        partial_amax = jnp.float32(0.0)
        for c in range(N_CHUNKS):
            if c + 1 < N_CHUNKS:
                w_copy(c + 1, (c + 1) % 2).start()
            w_copy(c, c % 2).wait()
            ystrip = jnp.maximum(
                jnp.dot(x_ref[...], wbuf[c % 2],
                        preferred_element_type=jnp.float32),
                0.0,
            )
            y_ref[:, pl.ds(c * cw, cw)] = ystrip
            partial_amax = jnp.maximum(partial_amax, jnp.max(ystrip))

        for s in range(N_DEV):
            @pl.when(s == my_i)
            def _():
                amax_ref[s] = jnp.full((8, 128), partial_amax, jnp.float32)

        for s in range(N_DEV):
            @pl.when(s != my_i)
            def _():
                rdma = pltpu.make_async_remote_copy(
                    src_ref=amax_ref.at[my_i],
                    dst_ref=amax_ref.at[my_i],
                    send_sem=amax_send_sems.at[s],
                    recv_sem=amax_recv_sems.at[my_i],
                    device_id=(s,),
                    device_id_type=pl.DeviceIdType.MESH,
                )
                rdma.start()
        for s in range(N_DEV):
            @pl.when(s != my_i)
            def _():
                d = pltpu.make_async_remote_copy(
                    src_ref=amax_ref.at[s],
                    dst_ref=amax_ref.at[s],
                    send_sem=amax_send_sems.at[s],
                    recv_sem=amax_recv_sems.at[s],
                    device_id=(s,),
                    device_id_type=pl.DeviceIdType.MESH,
                )
                d.wait_recv()
                d.wait_send()

        gmax = jnp.max(amax_ref[...])
        scale = gmax / 127.0

        for s in range(N_DEV):
            q_ref[s] = jnp.clip(
                jnp.round(y_ref[:, pl.ds(s * n_per, n_per)] / scale),
                -127.0, 127.0,
            ).astype(jnp.int8)

            @pl.when(s != my_i)
            def _():
                rdma = pltpu.make_async_remote_copy(
                    src_ref=q_ref.at[s],
                    dst_ref=recv_ref.at[pl.ds(my_i * m_per, m_per), :],
                    send_sem=data_send_sems.at[s],
                    recv_sem=data_recv_sems.at[my_i],
                    device_id=(s,),
                    device_id_type=pl.DeviceIdType.MESH,
                )
                rdma.start()

        recv_ref[pl.ds(my_i * m_per, m_per), :] = q_ref[my_i]

        for s in range(N_DEV):
            @pl.when(s != my_i)
            def _():
                d = pltpu.make_async_remote_copy(
                    src_ref=q_ref.at[s],
                    dst_ref=recv_ref.at[pl.ds(s * m_per, m_per), :],
                    send_sem=data_send_sems.at[s],
                    recv_sem=data_recv_sems.at[s],
                    device_id=(s,),
                    device_id_type=pl.DeviceIdType.MESH,
                )
                d.wait_recv()
                d.wait_send()

        out_ref[...] = recv_ref[...].astype(jnp.float32) * scale

    return pl.pallas_call(
        body,
        out_shape=jax.ShapeDtypeStruct((N_DEV * m_per, n_per), jnp.float32),
        in_specs=[
            pl.BlockSpec(memory_space=pltpu.VMEM),
            pl.BlockSpec(memory_space=pl.ANY),
        ],
        out_specs=pl.BlockSpec(memory_space=pltpu.VMEM),
        scratch_shapes=[
            pltpu.VMEM((2, k, cw), jnp.float32),
            pltpu.VMEM((m_per, n), jnp.float32),
            pltpu.VMEM((N_DEV, m_per, n_per), jnp.int8),
            pltpu.VMEM((N_DEV * m_per, n_per), jnp.int8),
            pltpu.VMEM((N_DEV, 8, 128), jnp.float32),
            pltpu.SemaphoreType.DMA((2,)),
            pltpu.SemaphoreType.DMA((N_DEV,)),
            pltpu.SemaphoreType.DMA((N_DEV,)),
            pltpu.SemaphoreType.DMA((N_DEV,)),
            pltpu.SemaphoreType.DMA((N_DEV,)),
        ],
        compiler_params=pltpu.CompilerParams(
            vmem_limit_bytes=100 * 1024 * 1024,
            collective_id=0,
        ),
    )(x, w_mat)


# device time: 35169 ns/iter; 1.1890x vs baseline; 1.0071x over previous
import jax
import jax.numpy as jnp
from jax import lax
from jax.experimental import pallas as pl
from jax.experimental.pallas import tpu as pltpu

N_DEV = 16
N_CHUNKS = 8


def kernel(x, w_mat):
    m_per, k = x.shape
    _, n = w_mat.shape
    n_per = n // N_DEV
    cw = n // N_CHUNKS

    def body(x_hbm, w_hbm, out_hbm, x_ref, wbuf, y_ref, q_ref, recv_ref,
             stage_ref, amax_ref, xdma_sem, wdma_sems, out_sem,
             amax_send_sems, amax_recv_sems, data_send_sems, data_recv_sems):
        my_i = lax.axis_index("i")

        barrier_sem = pltpu.get_barrier_semaphore()
        for s in range(N_DEV):
            @pl.when(s != my_i)
            def _():
                pl.semaphore_signal(
                    barrier_sem, inc=1,
                    device_id=(s,), device_id_type=pl.DeviceIdType.MESH,
                )

        x_copy = pltpu.make_async_copy(x_hbm, x_ref, xdma_sem)
        x_copy.start()

        def w_copy(c, slot):
            return pltpu.make_async_copy(
                w_hbm.at[:, pl.ds(c * cw, cw)],
                wbuf.at[slot],
                wdma_sems.at[slot],
            )

        w_copy(0, 0).start()
        x_copy.wait()
        partial_amax = jnp.float32(0.0)
        for c in range(N_CHUNKS):
            if c + 1 < N_CHUNKS:
                w_copy(c + 1, (c + 1) % 2).start()
            w_copy(c, c % 2).wait()
            ystrip = jnp.maximum(
                jnp.dot(x_ref[...], wbuf[c % 2],
                        preferred_element_type=jnp.float32),
                0.0,
            )
            y_ref[:, pl.ds(c * cw, cw)] = ystrip
            partial_amax = jnp.maximum(partial_amax, jnp.max(ystrip))

        for s in range(N_DEV):
            @pl.when(s == my_i)
            def _():
                amax_ref[s] = jnp.full((8, 128), partial_amax, jnp.float32)

        pl.semaphore_wait(barrier_sem, N_DEV - 1)

        for s in range(N_DEV):
            @pl.when(s != my_i)
            def _():
                rdma = pltpu.make_async_remote_copy(
                    src_ref=amax_ref.at[my_i],
                    dst_ref=amax_ref.at[my_i],
                    send_sem=amax_send_sems.at[s],
                    recv_sem=amax_recv_sems.at[my_i],
                    device_id=(s,),
                    device_id_type=pl.DeviceIdType.MESH,
                )
                rdma.start()
        for s in range(N_DEV):
            @pl.when(s != my_i)
            def _():
                d = pltpu.make_async_remote_copy(
                    src_ref=amax_ref.at[s],
                    dst_ref=amax_ref.at[s],
                    send_sem=amax_send_sems.at[s],
                    recv_sem=amax_recv_sems.at[s],
                    device_id=(s,),
                    device_id_type=pl.DeviceIdType.MESH,
                )
                d.wait_recv()
                d.wait_send()

        gmax = jnp.max(amax_ref[...])
        scale = gmax / 127.0

        for s in range(N_DEV):
            q_ref[s] = jnp.clip(
                jnp.round(y_ref[:, pl.ds(s * n_per, n_per)] / scale),
                -127.0, 127.0,
            ).astype(jnp.int8)

            @pl.when(s != my_i)
            def _():
                rdma = pltpu.make_async_remote_copy(
                    src_ref=q_ref.at[s],
                    dst_ref=recv_ref.at[pl.ds(my_i * m_per, m_per), :],
                    send_sem=data_send_sems.at[s],
                    recv_sem=data_recv_sems.at[my_i],
                    device_id=(s,),
                    device_id_type=pl.DeviceIdType.MESH,
                )
                rdma.start()

        recv_ref[pl.ds(my_i * m_per, m_per), :] = q_ref[my_i]

        for s in range(N_DEV):
            @pl.when(s != my_i)
            def _():
                d = pltpu.make_async_remote_copy(
                    src_ref=q_ref.at[s],
                    dst_ref=recv_ref.at[pl.ds(s * m_per, m_per), :],
                    send_sem=data_send_sems.at[s],
                    recv_sem=data_recv_sems.at[s],
                    device_id=(s,),
                    device_id_type=pl.DeviceIdType.MESH,
                )
                d.wait_recv()
                d.wait_send()

        stage_ref[...] = recv_ref[...].astype(jnp.float32) * scale
        out_copy = pltpu.make_async_copy(stage_ref, out_hbm, out_sem)
        out_copy.start()
        out_copy.wait()

    return pl.pallas_call(
        body,
        out_shape=jax.ShapeDtypeStruct((N_DEV * m_per, n_per), jnp.float32),
        in_specs=[
            pl.BlockSpec(memory_space=pl.ANY),
            pl.BlockSpec(memory_space=pl.ANY),
        ],
        out_specs=pl.BlockSpec(memory_space=pl.ANY),
        scratch_shapes=[
            pltpu.VMEM((m_per, k), jnp.float32),
            pltpu.VMEM((2, k, cw), jnp.float32),
            pltpu.VMEM((m_per, n), jnp.float32),
            pltpu.VMEM((N_DEV, m_per, n_per), jnp.int8),
            pltpu.VMEM((N_DEV * m_per, n_per), jnp.int8),
            pltpu.VMEM((N_DEV * m_per, n_per), jnp.float32),
            pltpu.VMEM((N_DEV, 8, 128), jnp.float32),
            pltpu.SemaphoreType.DMA,
            pltpu.SemaphoreType.DMA((2,)),
            pltpu.SemaphoreType.DMA,
            pltpu.SemaphoreType.DMA((N_DEV,)),
            pltpu.SemaphoreType.DMA((N_DEV,)),
            pltpu.SemaphoreType.DMA((N_DEV,)),
            pltpu.SemaphoreType.DMA((N_DEV,)),
        ],
        compiler_params=pltpu.CompilerParams(
            vmem_limit_bytes=100 * 1024 * 1024,
            collective_id=0,
        ),
    )(x, w_mat)


# device time: 32392 ns/iter; 1.2909x vs baseline; 1.0857x over previous
import jax
import jax.numpy as jnp
from jax import lax
from jax.experimental import pallas as pl
from jax.experimental.pallas import tpu as pltpu

N_DEV = 16
N_CHUNKS = 8


def kernel(x, w_mat):
    m_per, k = x.shape
    _, n = w_mat.shape
    n_per = n // N_DEV
    cw = n // N_CHUNKS

    def body(x_hbm, w_hbm, out_hbm, x_ref, wbuf, send_ref, recv_ref,
             stage_ref, amax_ref, xdma_sem, wdma_sems, out_sem,
             amax_send_sems, amax_recv_sems, data_send_sems, data_recv_sems):
        my_i = lax.axis_index("i")
        offset = my_i // 2

        barrier_sem = pltpu.get_barrier_semaphore()
        for s in range(N_DEV):
            @pl.when(s != my_i)
            def _():
                pl.semaphore_signal(
                    barrier_sem, inc=1,
                    device_id=(s,), device_id_type=pl.DeviceIdType.MESH,
                )

        x_copy = pltpu.make_async_copy(x_hbm, x_ref, xdma_sem)
        x_copy.start()

        def w_copy(c, slot):
            return pltpu.make_async_copy(
                w_hbm.at[:, pl.ds(c * cw, cw)],
                wbuf.at[slot],
                wdma_sems.at[slot],
            )

        def chunk_of(t):
            return (t + offset) % N_CHUNKS

        w_copy(chunk_of(0), 0).start()
        x_copy.wait()
        pl.semaphore_wait(barrier_sem, N_DEV - 1)

        partial_amax = jnp.float32(0.0)
        for t in range(N_CHUNKS):
            c = chunk_of(t)
            if t + 1 < N_CHUNKS:
                w_copy(chunk_of(t + 1), (t + 1) % 2).start()
            w_copy(c, t % 2).wait()
            ystrip = jnp.maximum(
                jnp.dot(x_ref[...], wbuf[t % 2],
                        preferred_element_type=jnp.float32),
                0.0,
            )
            partial_amax = jnp.maximum(partial_amax, jnp.max(ystrip))
            for j in range(2):
                s = 2 * c + j
                blk = ystrip[:, j * n_per:(j + 1) * n_per].astype(jnp.bfloat16)
                send_ref[s] = blk

                @pl.when(s == my_i)
                def _():
                    recv_ref[my_i] = blk

                @pl.when(s != my_i)
                def _():
                    rdma = pltpu.make_async_remote_copy(
                        src_ref=send_ref.at[s],
                        dst_ref=recv_ref.at[my_i],
                        send_sem=data_send_sems.at[s],
                        recv_sem=data_recv_sems.at[my_i],
                        device_id=(s,),
                        device_id_type=pl.DeviceIdType.MESH,
                    )
                    rdma.start()

        for s in range(N_DEV):
            @pl.when(s == my_i)
            def _():
                amax_ref[s] = jnp.full((8, 128), partial_amax, jnp.float32)
        for s in range(N_DEV):
            @pl.when(s != my_i)
            def _():
                rdma = pltpu.make_async_remote_copy(
                    src_ref=amax_ref.at[my_i],
                    dst_ref=amax_ref.at[my_i],
                    send_sem=amax_send_sems.at[s],
                    recv_sem=amax_recv_sems.at[my_i],
                    device_id=(s,),
                    device_id_type=pl.DeviceIdType.MESH,
                )
                rdma.start()
        for s in range(N_DEV):
            @pl.when(s != my_i)
            def _():
                d = pltpu.make_async_remote_copy(
                    src_ref=amax_ref.at[s],
                    dst_ref=amax_ref.at[s],
                    send_sem=amax_send_sems.at[s],
                    recv_sem=amax_recv_sems.at[s],
                    device_id=(s,),
                    device_id_type=pl.DeviceIdType.MESH,
                )
                d.wait_recv()
                d.wait_send()

        gmax = jnp.max(amax_ref[...])
        scale = gmax / 127.0

        for s in range(N_DEV):
            @pl.when(s != my_i)
            def _():
                d = pltpu.make_async_remote_copy(
                    src_ref=send_ref.at[s],
                    dst_ref=recv_ref.at[s],
                    send_sem=data_send_sems.at[s],
                    recv_sem=data_recv_sems.at[s],
                    device_id=(s,),
                    device_id_type=pl.DeviceIdType.MESH,
                )
                d.wait_recv()
                d.wait_send()
        for s in range(N_DEV):
            stage_ref[pl.ds(s * m_per, m_per), :] = jnp.clip(
                jnp.round(recv_ref[s].astype(jnp.float32) / scale),
                -127.0, 127.0,
            ) * scale

        out_copy = pltpu.make_async_copy(stage_ref, out_hbm, out_sem)
        out_copy.start()
        out_copy.wait()

    return pl.pallas_call(
        body,
        out_shape=jax.ShapeDtypeStruct((N_DEV * m_per, n_per), jnp.float32),
        in_specs=[
            pl.BlockSpec(memory_space=pl.ANY),
            pl.BlockSpec(memory_space=pl.ANY),
        ],
        out_specs=pl.BlockSpec(memory_space=pl.ANY),
        scratch_shapes=[
            pltpu.VMEM((m_per, k), jnp.float32),
            pltpu.VMEM((2, k, cw), jnp.float32),
            pltpu.VMEM((N_DEV, m_per, n_per), jnp.bfloat16),
            pltpu.VMEM((N_DEV, m_per, n_per), jnp.bfloat16),
            pltpu.VMEM((N_DEV * m_per, n_per), jnp.float32),
            pltpu.VMEM((N_DEV, 8, 128), jnp.float32),
            pltpu.SemaphoreType.DMA,
            pltpu.SemaphoreType.DMA((2,)),
            pltpu.SemaphoreType.DMA,
            pltpu.SemaphoreType.DMA((N_DEV,)),
            pltpu.SemaphoreType.DMA((N_DEV,)),
            pltpu.SemaphoreType.DMA((N_DEV,)),
            pltpu.SemaphoreType.DMA((N_DEV,)),
        ],
        compiler_params=pltpu.CompilerParams(
            vmem_limit_bytes=100 * 1024 * 1024,
            collective_id=0,
        ),
    )(x, w_mat)
